# baseline (speedup 1.0000x reference)
import functools

import jax
import jax.numpy as jnp
from jax.experimental import pallas as pl

B, DIN = 16384, 768
H1, H2, D = 512, 256, 64
L, K = 3, 1024

BB = 512


def _fused_kernel(x_ref, w1_ref, b1_ref, w2_ref, b2_ref, w3_ref, b3_ref,
                  cb_ref, ids_ref, quant_ref):
    x = x_ref[...]
    h = jnp.dot(x, w1_ref[...], preferred_element_type=jnp.float32)
    h = jnp.maximum(h + b1_ref[...], 0.0)
    h = jnp.dot(h, w2_ref[...], preferred_element_type=jnp.float32)
    h = jnp.maximum(h + b2_ref[...], 0.0)
    z = jnp.dot(h, w3_ref[...], preferred_element_type=jnp.float32)
    z = z + b3_ref[...]

    res = z
    quant = jnp.zeros_like(z)
    lane = jax.lax.broadcasted_iota(jnp.int32, (BB, K), 1)
    ids = []
    for l in range(L):
        cb = cb_ref[l]
        cn = jnp.sum(cb * cb, axis=1, keepdims=True).T
        rn = jnp.sum(res * res, axis=1, keepdims=True)
        dot = jax.lax.dot_general(res, cb, (((1,), (1,)), ((), ())),
                                  preferred_element_type=jnp.float32)
        d2 = rn - 2.0 * dot + cn
        mn = jnp.min(d2, axis=1, keepdims=True)
        idx = jnp.min(jnp.where(d2 == mn, lane, K), axis=1, keepdims=True)
        onehot = (lane == idx).astype(jnp.float32)
        sel = jnp.dot(onehot, cb, preferred_element_type=jnp.float32,
                      precision=jax.lax.Precision.HIGHEST)
        quant = quant + sel
        res = res - sel
        ids.append(idx)

    ids_ref[...] = jnp.concatenate(ids, axis=1)
    quant_ref[...] = quant


@jax.jit
def kernel(x, W1, b1, W2, b2, W3, b3, codebooks):
    grid = (B // BB,)
    full = lambda *shape: pl.BlockSpec(shape, lambda i: (0,) * len(shape))
    sem_ids, quant = pl.pallas_call(
        _fused_kernel,
        grid=grid,
        in_specs=[
            pl.BlockSpec((BB, DIN), lambda i: (i, 0)),
            full(DIN, H1),
            full(1, H1),
            full(H1, H2),
            full(1, H2),
            full(H2, D),
            full(1, D),
            full(L, K, D),
        ],
        out_specs=[
            pl.BlockSpec((BB, L), lambda i: (i, 0)),
            pl.BlockSpec((BB, D), lambda i: (i, 0)),
        ],
        out_shape=[
            jax.ShapeDtypeStruct((B, L), jnp.int32),
            jax.ShapeDtypeStruct((B, D), jnp.float32),
        ],
    )(x, W1, b1.reshape(1, H1), W2, b2.reshape(1, H2), W3, b3.reshape(1, D),
      codebooks)
    return sem_ids, quant


# speedup vs baseline: 2.1789x; 2.1789x over previous
import functools

import jax
import jax.numpy as jnp
from jax.experimental import pallas as pl

B, DIN = 16384, 768
H1, H2, D = 512, 256, 64
L, K = 3, 1024

BB = 512


def _fused_kernel(x_ref, w1_ref, b1_ref, w2_ref, b2_ref, w3_ref, b3_ref,
                  cb_ref, ids_ref, quant_ref):
    x = x_ref[...]
    h = jnp.dot(x, w1_ref[...], preferred_element_type=jnp.float32)
    h = jnp.maximum(h + b1_ref[...], 0.0)
    h = jnp.dot(h, w2_ref[...], preferred_element_type=jnp.float32)
    h = jnp.maximum(h + b2_ref[...], 0.0)
    z = jnp.dot(h, w3_ref[...], preferred_element_type=jnp.float32)
    z = z + b3_ref[...]

    res = z
    quant = jnp.zeros_like(z)
    lane = jax.lax.broadcasted_iota(jnp.int32, (BB, K), 1)
    ids = []
    for l in range(L):
        cb = cb_ref[l]
        cn = jnp.sum(cb * cb, axis=1, keepdims=True).T
        rn = jnp.sum(res * res, axis=1, keepdims=True)
        dot = jax.lax.dot_general(res, cb, (((1,), (1,)), ((), ())),
                                  preferred_element_type=jnp.float32)
        d2 = rn - 2.0 * dot + cn
        mn = jnp.min(d2, axis=1, keepdims=True)
        idx = jnp.min(jnp.where(d2 == mn, lane, K), axis=1, keepdims=True)
        onehot = (lane == idx).astype(jnp.float32)
        sel = jnp.dot(onehot, cb, preferred_element_type=jnp.float32)
        quant = quant + sel
        res = res - sel
        ids.append(idx)

    ids_ref[...] = jnp.concatenate(ids, axis=1)
    quant_ref[...] = quant


@jax.jit
def kernel(x, W1, b1, W2, b2, W3, b3, codebooks):
    grid = (B // BB,)
    full = lambda *shape: pl.BlockSpec(shape, lambda i: (0,) * len(shape))
    sem_ids, quant = pl.pallas_call(
        _fused_kernel,
        grid=grid,
        in_specs=[
            pl.BlockSpec((BB, DIN), lambda i: (i, 0)),
            full(DIN, H1),
            full(1, H1),
            full(H1, H2),
            full(1, H2),
            full(H2, D),
            full(1, D),
            full(L, K, D),
        ],
        out_specs=[
            pl.BlockSpec((BB, L), lambda i: (i, 0)),
            pl.BlockSpec((BB, D), lambda i: (i, 0)),
        ],
        out_shape=[
            jax.ShapeDtypeStruct((B, L), jnp.int32),
            jax.ShapeDtypeStruct((B, D), jnp.float32),
        ],
    )(x, W1, b1.reshape(1, H1), W2, b2.reshape(1, H2), W3, b3.reshape(1, D),
      codebooks)
    return sem_ids, quant
